# baseline (speedup 1.0000x reference)
import functools

import jax
import jax.numpy as jnp
from jax.experimental import pallas as pl

_N = 65536
_C = 256
_K = 128
_BR = 512


def _rowsum(x, ones_c):
    return jax.lax.dot_general(x, ones_c, (((1,), (0,)), ((), ())),
                               preferred_element_type=jnp.float32)


def _stage1_body(s_ref, lab_ref, met_ref, cfg_ref, cbg_ref):
    s = s_ref[...]
    lab = lab_ref[...]

    col = jax.lax.broadcasted_iota(jnp.int32, s.shape, 1)
    m = jnp.max(s, axis=1, keepdims=True)
    e = jnp.exp(s - m)

    ones_c = jnp.ones((_C + 1, 1), jnp.float32)
    sum_e = _rowsum(e, ones_c)

    keep = col != (_C - 1)
    m2 = jnp.max(jnp.where(keep, s, -jnp.inf), axis=1, keepdims=True)
    un = jnp.where(keep, jnp.exp(s - m2), 0.0)
    z2 = jnp.sum(un, axis=1, keepdims=True)
    p = un / z2
    lp = jnp.where(p > 0, jnp.log(p), 0.0)
    met = -jnp.sum(p * lp, axis=1, keepdims=True)
    met_ref[...] = jax.lax.transpose(met, (1, 0))[None]

    e_lab = _rowsum(jnp.where(col == lab, e, 0.0), ones_c)
    s_masked = sum_e - e_lab
    gt = e_lab / sum_e
    w = gt * (1.0 - gt)
    lse_m = m + jnp.log(s_masked)

    s254 = s[:, _C - 2:_C - 1]
    s255 = s[:, _C - 1:_C]
    s256 = s[:, _C:_C + 1]
    v_fg = jnp.where(lab >= _C - 1, s254, s255)
    v_bg = jnp.where(lab == _C, s255, s256)
    cfg_ref[...] = jax.lax.transpose(w * (v_fg - lse_m), (1, 0))[None]
    cbg_ref[...] = jax.lax.transpose(w * (v_bg - lse_m), (1, 0))[None]


def _stage2_body(met_ref, lab_ref, cfg_ref, cbg_ref, out_ref):
    met = met_ref[...]
    lab = lab_ref[...]
    u = jax.lax.bitcast_convert_type(met, jnp.uint32)
    sgn = (u >> jnp.uint32(31)) > jnp.uint32(0)
    key = jnp.where(sgn, ~u, u | jnp.uint32(0x80000000))
    rows = met.shape[1]
    idx = (jax.lax.broadcasted_iota(jnp.int32, met.shape, 0) * rows
           + jax.lax.broadcasted_iota(jnp.int32, met.shape, 1))
    fg = lab != _C
    zero = jnp.uint32(0)
    kp = jnp.where(fg, key, zero)
    kn = jnp.where(fg, zero, key)

    one = jnp.ones((1, 1), jnp.uint32)
    kvec = jnp.full((1, 1), _K, jnp.int32)

    pp = jnp.zeros((1, 1), jnp.uint32)
    pn = jnp.zeros((1, 1), jnp.uint32)
    for b in range(31, -1, -1):
        qp = pp | (one << b)
        qn = pn | (one << b)
        cp = jnp.sum((kp >= qp).astype(jnp.int32), keepdims=True)
        cn = jnp.sum((kn >= qn).astype(jnp.int32), keepdims=True)
        pp = jnp.where(cp >= kvec, qp, pp)
        pn = jnp.where(cn >= kvec, qn, pn)

    ep = kvec - jnp.sum((kp > pp).astype(jnp.int32), keepdims=True)
    en = kvec - jnp.sum((kn > pn).astype(jnp.int32), keepdims=True)
    tp = kp == pp
    tn = kn == pn
    ione = jnp.ones((1, 1), jnp.int32)
    cutp = jnp.zeros((1, 1), jnp.int32)
    cutn = jnp.zeros((1, 1), jnp.int32)
    for b in range(16, -1, -1):
        qp = cutp + (ione << b)
        qn = cutn + (ione << b)
        cp = jnp.sum((tp & (idx < qp)).astype(jnp.int32), keepdims=True)
        cn = jnp.sum((tn & (idx < qn)).astype(jnp.int32), keepdims=True)
        cutp = jnp.where(cp <= ep, qp, cutp)
        cutn = jnp.where(cn <= en, qn, cutn)

    selp = (kp > pp) | (tp & (idx < cutp))
    seln = (kn > pn) | (tn & (idx < cutn))
    s = (jnp.sum(jnp.where(selp, cfg_ref[...], 0.0))
         + jnp.sum(jnp.where(seln, cbg_ref[...], 0.0)))
    out_ref[...] = jnp.reshape(-s / jnp.float32(2 * _K), (1, 1))


@jax.jit
def kernel(scores, labels):
    lab2 = labels.reshape(_N, 1)

    grid = _N // _BR
    met, cfg, cbg = pl.pallas_call(
        _stage1_body,
        grid=(grid,),
        in_specs=[
            pl.BlockSpec((_BR, _C + 1), lambda i: (i, 0)),
            pl.BlockSpec((_BR, 1), lambda i: (i, 0)),
        ],
        out_specs=[
            pl.BlockSpec((1, 1, _BR), lambda i: (i, 0, 0)),
            pl.BlockSpec((1, 1, _BR), lambda i: (i, 0, 0)),
            pl.BlockSpec((1, 1, _BR), lambda i: (i, 0, 0)),
        ],
        out_shape=[jax.ShapeDtypeStruct((_N // _BR, 1, _BR), jnp.float32)] * 3,
    )(scores, lab2)

    r, c = _N // 128, 128
    out = pl.pallas_call(
        _stage2_body,
        out_shape=jax.ShapeDtypeStruct((1, 1), jnp.float32),
    )(met.reshape(r, c), labels.reshape(r, c),
      cfg.reshape(r, c), cbg.reshape(r, c))
    return out[0, 0]


# speedup vs baseline: 1.1558x; 1.1558x over previous
import functools

import jax
import jax.numpy as jnp
from jax.experimental import pallas as pl

_N = 65536
_C = 256
_K = 128
_BR = 2048


def _rowsum(x, ones_c):
    return jax.lax.dot_general(x, ones_c, (((1,), (0,)), ((), ())),
                               preferred_element_type=jnp.float32)


def _stage1_body(s_ref, lab_ref, met_ref, cfg_ref, cbg_ref):
    s = s_ref[...]
    lab = lab_ref[...]

    col = jax.lax.broadcasted_iota(jnp.int32, s.shape, 1)

    keep = col != (_C - 1)
    m2 = jnp.max(jnp.where(keep, s, -jnp.inf), axis=1, keepdims=True)
    E = jnp.exp(s - m2)
    un = jnp.where(keep, E, 0.0)
    z2 = jnp.sum(un, axis=1, keepdims=True)
    p = un / z2
    lp = jnp.where(p > 0, jnp.log(p), 0.0)
    met = -jnp.sum(p * lp, axis=1, keepdims=True)
    met_ref[...] = jax.lax.transpose(met, (1, 0))[None]

    ones_c = jnp.ones((_C + 1, 1), jnp.float32)
    sum_e = _rowsum(E, ones_c)
    e_lab = _rowsum(jnp.where(col == lab, E, 0.0), ones_c)
    s_masked = sum_e - e_lab
    gt = e_lab / sum_e
    w = gt * (1.0 - gt)
    lse_m = m2 + jnp.log(s_masked)

    s254 = s[:, _C - 2:_C - 1]
    s255 = s[:, _C - 1:_C]
    s256 = s[:, _C:_C + 1]
    v_fg = jnp.where(lab >= _C - 1, s254, s255)
    v_bg = jnp.where(lab == _C, s255, s256)
    cfg_ref[...] = jax.lax.transpose(w * (v_fg - lse_m), (1, 0))[None]
    cbg_ref[...] = jax.lax.transpose(w * (v_bg - lse_m), (1, 0))[None]


def _stage2_body(met_ref, lab_ref, cfg_ref, cbg_ref, out_ref):
    met = met_ref[...]
    lab = lab_ref[...]
    u = jax.lax.bitcast_convert_type(met, jnp.uint32)
    sgn = (u >> jnp.uint32(31)) > jnp.uint32(0)
    key = jnp.where(sgn, ~u, u | jnp.uint32(0x80000000))
    rows = met.shape[1]
    idx = (jax.lax.broadcasted_iota(jnp.int32, met.shape, 0) * rows
           + jax.lax.broadcasted_iota(jnp.int32, met.shape, 1))
    fg = lab != _C
    zero = jnp.uint32(0)
    kp = jnp.where(fg, key, zero)
    kn = jnp.where(fg, zero, key)

    one = jnp.ones((1, 1), jnp.uint32)
    kvec = jnp.full((1, 1), _K, jnp.int32)

    pp = jnp.zeros((1, 1), jnp.uint32)
    pn = jnp.zeros((1, 1), jnp.uint32)
    for b in range(31, -1, -1):
        qp = pp | (one << b)
        qn = pn | (one << b)
        cp = jnp.sum((kp >= qp).astype(jnp.int32), keepdims=True)
        cn = jnp.sum((kn >= qn).astype(jnp.int32), keepdims=True)
        pp = jnp.where(cp >= kvec, qp, pp)
        pn = jnp.where(cn >= kvec, qn, pn)

    ep = kvec - jnp.sum((kp > pp).astype(jnp.int32), keepdims=True)
    en = kvec - jnp.sum((kn > pn).astype(jnp.int32), keepdims=True)
    tp = kp == pp
    tn = kn == pn
    ione = jnp.ones((1, 1), jnp.int32)
    cutp = jnp.zeros((1, 1), jnp.int32)
    cutn = jnp.zeros((1, 1), jnp.int32)
    for b in range(16, -1, -1):
        qp = cutp + (ione << b)
        qn = cutn + (ione << b)
        cp = jnp.sum((tp & (idx < qp)).astype(jnp.int32), keepdims=True)
        cn = jnp.sum((tn & (idx < qn)).astype(jnp.int32), keepdims=True)
        cutp = jnp.where(cp <= ep, qp, cutp)
        cutn = jnp.where(cn <= en, qn, cutn)

    selp = (kp > pp) | (tp & (idx < cutp))
    seln = (kn > pn) | (tn & (idx < cutn))
    s = (jnp.sum(jnp.where(selp, cfg_ref[...], 0.0))
         + jnp.sum(jnp.where(seln, cbg_ref[...], 0.0)))
    out_ref[...] = jnp.reshape(-s / jnp.float32(2 * _K), (1, 1))


@jax.jit
def kernel(scores, labels):
    lab2 = labels.reshape(_N, 1)

    grid = _N // _BR
    met, cfg, cbg = pl.pallas_call(
        _stage1_body,
        grid=(grid,),
        in_specs=[
            pl.BlockSpec((_BR, _C + 1), lambda i: (i, 0)),
            pl.BlockSpec((_BR, 1), lambda i: (i, 0)),
        ],
        out_specs=[
            pl.BlockSpec((1, 1, _BR), lambda i: (i, 0, 0)),
            pl.BlockSpec((1, 1, _BR), lambda i: (i, 0, 0)),
            pl.BlockSpec((1, 1, _BR), lambda i: (i, 0, 0)),
        ],
        out_shape=[jax.ShapeDtypeStruct((_N // _BR, 1, _BR), jnp.float32)] * 3,
    )(scores, lab2)

    r, c = _N // 128, 128
    out = pl.pallas_call(
        _stage2_body,
        out_shape=jax.ShapeDtypeStruct((1, 1), jnp.float32),
    )(met.reshape(r, c), labels.reshape(r, c),
      cfg.reshape(r, c), cbg.reshape(r, c))
    return out[0, 0]
